# Initial kernel scaffold; baseline (speedup 1.0000x reference)
import jax
import jax.numpy as jnp
from jax.experimental import pallas as pl


def kernel(x, edge_index, W1, att_src1, att_dst1, b1, bn1_g, bn1_b, bn1_rm, bn1_rv, W2, att_src2, att_dst2, b2, bn2_g, bn2_b, bn2_rm, bn2_rv, W3, att_src3, att_dst3, b3, bn3_g, bn3_b, bn3_rm, bn3_rv, Wc, bc):
    raise NotImplementedError("write your pallas kernel here")


# speedup vs baseline: 13.5671x; 13.5671x over previous
import functools

import jax
import jax.numpy as jnp
from jax import lax
from jax.experimental import pallas as pl
from jax.experimental.pallas import tpu as pltpu
from jax.experimental.pallas import tpu_sc as plsc

N = 10000
E = 320000
D = 128
NC = 2
NS = 16
NW = NC * NS
BLK = 400


def _sc_mesh():
    return plsc.VectorSubcoreMesh(
        core_axis_name="c", subcore_axis_name="s", num_cores=NC, num_subcores=NS
    )


def _dense_in_body(x_ref, w_ref, ms_ref, md_ref, hh_ref, as_ref, ad_ref):
    hb = jnp.dot(x_ref[...], w_ref[...], preferred_element_type=jnp.float32)
    for h in range(4):
        hh_ref[h] = hb[:, h * D:(h + 1) * D]
    as_ref[...] = jnp.dot(hb, ms_ref[...], preferred_element_type=jnp.float32)
    ad_ref[...] = jnp.dot(hb, md_ref[...], preferred_element_type=jnp.float32)


def _dense_in(x, w, ms, md):
    return pl.pallas_call(
        _dense_in_body,
        grid=(N // BLK,),
        in_specs=[
            pl.BlockSpec((BLK, D), lambda i: (i, 0)),
            pl.BlockSpec((D, 4 * D), lambda i: (0, 0)),
            pl.BlockSpec((4 * D, 4), lambda i: (0, 0)),
            pl.BlockSpec((4 * D, 4), lambda i: (0, 0)),
        ],
        out_specs=[
            pl.BlockSpec((4, BLK, D), lambda i: (0, i, 0)),
            pl.BlockSpec((BLK, 4), lambda i: (i, 0)),
            pl.BlockSpec((BLK, 4), lambda i: (i, 0)),
        ],
        out_shape=[
            jax.ShapeDtypeStruct((4, N, D), jnp.float32),
            jax.ShapeDtypeStruct((N, 4), jnp.float32),
            jax.ShapeDtypeStruct((N, 4), jnp.float32),
        ],
    )(x, w, ms, md)


def _make_finish_body(nh_next):
    def body(un_ref, den_ref, sc_ref, sh_ref, w_ref, ms_ref, md_ref,
             hh_ref, as_ref, ad_ref):
        den = jnp.sum(den_ref[...], axis=0)
        parts = [un_ref[h] / (den[:, h:h + 1] + 1e-16) for h in range(4)]
        xb = jnp.concatenate(parts, axis=1)
        xb = xb * sc_ref[...] + sh_ref[...]
        xb = jnp.where(xb > 0, xb, jnp.exp(xb) - 1.0)
        hb = jnp.dot(xb, w_ref[...], preferred_element_type=jnp.float32)
        if nh_next == 4:
            for h in range(4):
                hh_ref[h] = hb[:, h * D:(h + 1) * D]
        else:
            hh_ref[...] = hb
        as_ref[...] = jnp.dot(hb, ms_ref[...], preferred_element_type=jnp.float32)
        ad_ref[...] = jnp.dot(hb, md_ref[...], preferred_element_type=jnp.float32)
    return body


def _finish(un, den_parts, sc, sh, w, ms, md, nh_next):
    wcols = w.shape[1]
    if nh_next == 4:
        hh_spec = pl.BlockSpec((4, BLK, D), lambda i: (0, i, 0))
        hh_shape = jax.ShapeDtypeStruct((4, N, D), jnp.float32)
    else:
        hh_spec = pl.BlockSpec((BLK, D), lambda i: (i, 0))
        hh_shape = jax.ShapeDtypeStruct((N, D), jnp.float32)
    return pl.pallas_call(
        _make_finish_body(nh_next),
        grid=(N // BLK,),
        in_specs=[
            pl.BlockSpec((4, BLK, D), lambda i: (0, i, 0)),
            pl.BlockSpec((NW, BLK, 4), lambda i: (0, i, 0)),
            pl.BlockSpec((1, 4 * D), lambda i: (0, 0)),
            pl.BlockSpec((1, 4 * D), lambda i: (0, 0)),
            pl.BlockSpec((4 * D, wcols), lambda i: (0, 0)),
            pl.BlockSpec((wcols, nh_next), lambda i: (0, 0)),
            pl.BlockSpec((wcols, nh_next), lambda i: (0, 0)),
        ],
        out_specs=[
            hh_spec,
            pl.BlockSpec((BLK, nh_next), lambda i: (i, 0)),
            pl.BlockSpec((BLK, nh_next), lambda i: (i, 0)),
        ],
        out_shape=[
            hh_shape,
            jax.ShapeDtypeStruct((N, nh_next), jnp.float32),
            jax.ShapeDtypeStruct((N, nh_next), jnp.float32),
        ],
    )(un, den_parts, sc, sh, w, ms, md)


def _final_body(un_ref, den_ref, sc_ref, sh_ref, wc_ref, bc_ref, out_ref):
    u = un_ref[0] + un_ref[1]
    den = jnp.sum(den_ref[...], axis=0)
    x = u / (den + 1e-16)
    x = x * sc_ref[...] + sh_ref[...]
    x = jnp.where(x > 0, x, jnp.exp(x) - 1.0)
    out_ref[...] = jnp.dot(x, wc_ref[...], preferred_element_type=jnp.float32) + bc_ref[...]


def _final(un_parts, den_parts, sc, sh, wc, bc):
    return pl.pallas_call(
        _final_body,
        grid=(N // BLK,),
        in_specs=[
            pl.BlockSpec((2, BLK, D), lambda i: (0, i, 0)),
            pl.BlockSpec((NW, BLK, 1), lambda i: (0, i, 0)),
            pl.BlockSpec((1, D), lambda i: (0, 0)),
            pl.BlockSpec((1, D), lambda i: (0, 0)),
            pl.BlockSpec((D, 2), lambda i: (0, 0)),
            pl.BlockSpec((1, 2), lambda i: (0, 0)),
        ],
        out_specs=pl.BlockSpec((BLK, 2), lambda i: (i, 0)),
        out_shape=jax.ShapeDtypeStruct((N, 2), jnp.float32),
    )(un_parts, den_parts, sc, sh, wc, bc)


CHK_A = 400


def _make_softmax(H):
    HN = H * N
    EPT = E // NW

    def body(as_hbm, ad_hbm, src_hbm, dst_hbm, ex_hbm, den_hbm,
             as_v, ad_v, den_v, src_v, dst_v, ex_v):
        core = lax.axis_index("c")
        sub = lax.axis_index("s")
        wid = core * NS + sub
        pltpu.sync_copy(as_hbm, as_v)
        pltpu.sync_copy(ad_hbm, ad_v)
        z16 = jnp.zeros((16,), jnp.float32)

        def zbody(i, _):
            den_v[pl.ds(i * 16, 16)] = z16
            return 0
        lax.fori_loop(0, HN // 16, zbody, 0)

        ebase = wid * EPT

        def chunk(kk, _):
            base = ebase + kk * CHK_A
            pltpu.sync_copy(src_hbm.at[pl.ds(base, CHK_A)], src_v)
            pltpu.sync_copy(dst_hbm.at[pl.ds(base, CHK_A)], dst_v)

            def grp(g, _):
                sv = src_v[pl.ds(g * 16, 16)] * H
                dv = dst_v[pl.ds(g * 16, 16)] * H
                for h in range(H):
                    a = plsc.load_gather(as_v, [sv + h])
                    b = plsc.load_gather(ad_v, [dv + h])
                    e = a + b
                    e = jnp.where(e >= 0, e, 0.2 * e)
                    ex = jnp.exp(e)
                    ex_v[pl.ds(h * CHK_A + g * 16, 16)] = ex
                    plsc.addupdate_scatter(den_v, [dv + h], ex)
                return 0
            lax.fori_loop(0, CHK_A // 16, grp, 0)
            for h in range(H):
                pltpu.sync_copy(ex_v.at[pl.ds(h * CHK_A, CHK_A)],
                                ex_hbm.at[pl.ds(h * E + base, CHK_A)])
            return 0
        lax.fori_loop(0, EPT // CHK_A, chunk, 0)
        pltpu.sync_copy(den_v, den_hbm.at[pl.ds(wid * HN, HN)])

    return pl.kernel(
        body,
        out_type=[
            jax.ShapeDtypeStruct((H * E,), jnp.float32),
            jax.ShapeDtypeStruct((NW * HN,), jnp.float32),
        ],
        mesh=_sc_mesh(),
        compiler_params=pltpu.CompilerParams(needs_layout_passes=False),
        scratch_types=[
            pltpu.VMEM((HN,), jnp.float32),
            pltpu.VMEM((HN,), jnp.float32),
            pltpu.VMEM((HN,), jnp.float32),
            pltpu.VMEM((CHK_A,), jnp.int32),
            pltpu.VMEM((CHK_A,), jnp.int32),
            pltpu.VMEM((H * CHK_A,), jnp.float32),
        ],
    )


K_B = 80
ZR = 128
NP = 10240
NRT = NP // NS


def _make_agg(H):
    HPC = 2 if H == 4 else 1
    EPT = E // NS if H == 4 else E // NW
    out_rows = H * NP if H == 4 else NC * NP

    def body(hh_hbm, src_hbm, dst_hbm, ex_hbm, un_hbm,
             acc, zrow, src_v, dst_v, sidx_v, ex_v, rows_v, sem):
        core = lax.axis_index("c")
        sub = lax.axis_index("s")
        z16 = jnp.zeros((16,), jnp.float32)

        def zb(i, _):
            for cc in range(8):
                zrow[i, pl.ds(cc * 16, 16)] = z16
            return 0
        lax.fori_loop(0, ZR, zb, 0)

        r0 = sub * NRT
        for hh in range(HPC):
            if H == 4:
                h = core * HPC + hh
                ebase = sub * EPT
                orow = h * NP + r0
            else:
                h = 0
                ebase = (core * NS + sub) * EPT
                orow = core * NP + r0
            for q in range(NRT // ZR):
                pltpu.sync_copy(zrow, acc.at[pl.ds(r0 + q * ZR, ZR)])
            plsc.subcore_barrier()

            def chunk(kk, _):
                base = ebase + kk * K_B
                pltpu.sync_copy(src_hbm.at[pl.ds(base, K_B)], src_v)
                pltpu.sync_copy(dst_hbm.at[pl.ds(base, K_B)], dst_v)
                pltpu.sync_copy(ex_hbm.at[pl.ds(h * E + base, K_B)], ex_v)

                def adj(g, _):
                    sl = pl.ds(g * 16, 16)
                    sidx_v[sl] = src_v[sl] + h * N
                    return 0
                lax.fori_loop(0, K_B // 16, adj, 0)
                pltpu.async_copy(hh_hbm.at[sidx_v], rows_v, sem).wait()

                def scale(j, _):
                    eb = plsc.load_gather(ex_v, [jnp.full((16,), j, jnp.int32)])
                    for cc in range(8):
                        sl = pl.ds(cc * 16, 16)
                        rows_v[j, sl] = rows_v[j, sl] * eb
                    return 0
                lax.fori_loop(0, K_B, scale, 0)
                pltpu.sync_copy(rows_v, acc.at[dst_v], add=True)
                return 0
            lax.fori_loop(0, EPT // K_B, chunk, 0)
            plsc.subcore_barrier()
            pltpu.sync_copy(acc.at[pl.ds(r0, NRT)],
                            un_hbm.at[pl.ds(orow, NRT)])

    return pl.kernel(
        body,
        out_type=jax.ShapeDtypeStruct((out_rows, D), jnp.float32),
        mesh=_sc_mesh(),
        compiler_params=pltpu.CompilerParams(needs_layout_passes=False),
        scratch_types=[
            pltpu.VMEM_SHARED((NP, D), jnp.float32),
            pltpu.VMEM((ZR, D), jnp.float32),
            pltpu.VMEM((K_B,), jnp.int32),
            pltpu.VMEM((K_B,), jnp.int32),
            pltpu.VMEM((K_B,), jnp.int32),
            pltpu.VMEM((K_B,), jnp.float32),
            pltpu.VMEM((K_B, D), jnp.float32),
            pltpu.SemaphoreType.DMA,
        ],
    )


_softmax4 = _make_softmax(4)
_softmax1 = _make_softmax(1)
_agg4 = _make_agg(4)
_agg1 = _make_agg(1)


def _att_mat(att):
    H, C = att.shape
    mask = jnp.kron(jnp.eye(H, dtype=att.dtype), jnp.ones((C, 1), att.dtype))
    return mask * att.reshape(H * C, 1)


def _fold_bn(b, g, bb, rm, rv, eps=1e-5):
    s = g / jnp.sqrt(rv + eps)
    return s.reshape(1, -1), ((b - rm) * s + bb).reshape(1, -1)


def kernel(x, edge_index,
           W1, att_src1, att_dst1, b1, bn1_g, bn1_b, bn1_rm, bn1_rv,
           W2, att_src2, att_dst2, b2, bn2_g, bn2_b, bn2_rm, bn2_rv,
           W3, att_src3, att_dst3, b3, bn3_g, bn3_b, bn3_rm, bn3_rv,
           Wc, bc):
    src = edge_index[0]
    dst = edge_index[1]
    ms1, md1 = _att_mat(att_src1), _att_mat(att_dst1)
    ms2, md2 = _att_mat(att_src2), _att_mat(att_dst2)
    ms3, md3 = _att_mat(att_src3), _att_mat(att_dst3)
    sc1, sh1 = _fold_bn(b1, bn1_g, bn1_b, bn1_rm, bn1_rv)
    sc2, sh2 = _fold_bn(b2, bn2_g, bn2_b, bn2_rm, bn2_rv)
    sc3, sh3 = _fold_bn(b3, bn3_g, bn3_b, bn3_rm, bn3_rv)

    hh1, as1, ad1 = _dense_in(x, W1.T, ms1, md1)
    ex1, den1 = _softmax4(as1.reshape(-1), ad1.reshape(-1), src, dst)
    un1 = _agg4(hh1.reshape(4 * N, D), src, dst, ex1)
    hh2, as2, ad2 = _finish(un1.reshape(4, NP, D), den1.reshape(NW, N, 4),
                            sc1, sh1, W2.T, ms2, md2, 4)
    ex2, den2 = _softmax4(as2.reshape(-1), ad2.reshape(-1), src, dst)
    un2 = _agg4(hh2.reshape(4 * N, D), src, dst, ex2)
    hh3, as3, ad3 = _finish(un2.reshape(4, NP, D), den2.reshape(NW, N, 4),
                            sc2, sh2, W3.T, ms3, md3, 1)
    ex3, den3 = _softmax1(as3.reshape(-1), ad3.reshape(-1), src, dst)
    un3 = _agg1(hh3, src, dst, ex3)
    out = _final(un3.reshape(2, NP, D), den3.reshape(NW, N, 1),
                 sc3, sh3, Wc.T, bc.reshape(1, 2))
    return out


# speedup vs baseline: 18.4576x; 1.3605x over previous
import functools

import jax
import jax.numpy as jnp
from jax import lax
from jax.experimental import pallas as pl
from jax.experimental.pallas import tpu as pltpu
from jax.experimental.pallas import tpu_sc as plsc

N = 10000
E = 320000
D = 128
NC = 2
NS = 16
NW = NC * NS
BLK = 400


def _sc_mesh():
    return plsc.VectorSubcoreMesh(
        core_axis_name="c", subcore_axis_name="s", num_cores=NC, num_subcores=NS
    )


def _dense_in_body(x_ref, w_ref, ms_ref, md_ref, hh_ref, as_ref, ad_ref):
    hb = jnp.dot(x_ref[...], w_ref[...], preferred_element_type=jnp.float32)
    for h in range(4):
        hh_ref[h] = hb[:, h * D:(h + 1) * D]
    as_ref[...] = jnp.dot(hb, ms_ref[...], preferred_element_type=jnp.float32)
    ad_ref[...] = jnp.dot(hb, md_ref[...], preferred_element_type=jnp.float32)


def _dense_in(x, w, ms, md):
    return pl.pallas_call(
        _dense_in_body,
        grid=(N // BLK,),
        in_specs=[
            pl.BlockSpec((BLK, D), lambda i: (i, 0)),
            pl.BlockSpec((D, 4 * D), lambda i: (0, 0)),
            pl.BlockSpec((4 * D, 4), lambda i: (0, 0)),
            pl.BlockSpec((4 * D, 4), lambda i: (0, 0)),
        ],
        out_specs=[
            pl.BlockSpec((4, BLK, D), lambda i: (0, i, 0)),
            pl.BlockSpec((BLK, 4), lambda i: (i, 0)),
            pl.BlockSpec((BLK, 4), lambda i: (i, 0)),
        ],
        out_shape=[
            jax.ShapeDtypeStruct((4, N, D), jnp.float32),
            jax.ShapeDtypeStruct((N, 4), jnp.float32),
            jax.ShapeDtypeStruct((N, 4), jnp.float32),
        ],
    )(x, w, ms, md)


def _make_finish_body(nh_next):
    def body(un_ref, den_ref, sc_ref, sh_ref, w_ref, ms_ref, md_ref,
             hh_ref, as_ref, ad_ref):
        den = jnp.sum(den_ref[...], axis=0)
        parts = [un_ref[h] / (den[:, h:h + 1] + 1e-16) for h in range(4)]
        xb = jnp.concatenate(parts, axis=1)
        xb = xb * sc_ref[...] + sh_ref[...]
        xb = jnp.where(xb > 0, xb, jnp.exp(xb) - 1.0)
        hb = jnp.dot(xb, w_ref[...], preferred_element_type=jnp.float32)
        if nh_next == 4:
            for h in range(4):
                hh_ref[h] = hb[:, h * D:(h + 1) * D]
        else:
            hh_ref[...] = hb
        as_ref[...] = jnp.dot(hb, ms_ref[...], preferred_element_type=jnp.float32)
        ad_ref[...] = jnp.dot(hb, md_ref[...], preferred_element_type=jnp.float32)
    return body


def _finish(un, den_parts, sc, sh, w, ms, md, nh_next):
    wcols = w.shape[1]
    if nh_next == 4:
        hh_spec = pl.BlockSpec((4, BLK, D), lambda i: (0, i, 0))
        hh_shape = jax.ShapeDtypeStruct((4, N, D), jnp.float32)
    else:
        hh_spec = pl.BlockSpec((BLK, D), lambda i: (i, 0))
        hh_shape = jax.ShapeDtypeStruct((N, D), jnp.float32)
    return pl.pallas_call(
        _make_finish_body(nh_next),
        grid=(N // BLK,),
        in_specs=[
            pl.BlockSpec((4, BLK, D), lambda i: (0, i, 0)),
            pl.BlockSpec((NW, BLK, 4), lambda i: (0, i, 0)),
            pl.BlockSpec((1, 4 * D), lambda i: (0, 0)),
            pl.BlockSpec((1, 4 * D), lambda i: (0, 0)),
            pl.BlockSpec((4 * D, wcols), lambda i: (0, 0)),
            pl.BlockSpec((wcols, nh_next), lambda i: (0, 0)),
            pl.BlockSpec((wcols, nh_next), lambda i: (0, 0)),
        ],
        out_specs=[
            hh_spec,
            pl.BlockSpec((BLK, nh_next), lambda i: (i, 0)),
            pl.BlockSpec((BLK, nh_next), lambda i: (i, 0)),
        ],
        out_shape=[
            hh_shape,
            jax.ShapeDtypeStruct((N, nh_next), jnp.float32),
            jax.ShapeDtypeStruct((N, nh_next), jnp.float32),
        ],
    )(un, den_parts, sc, sh, w, ms, md)


def _final_body(un_ref, den_ref, sc_ref, sh_ref, wc_ref, bc_ref, out_ref):
    u = un_ref[0] + un_ref[1]
    den = jnp.sum(den_ref[...], axis=0)
    x = u / (den + 1e-16)
    x = x * sc_ref[...] + sh_ref[...]
    x = jnp.where(x > 0, x, jnp.exp(x) - 1.0)
    out_ref[...] = jnp.dot(x, wc_ref[...], preferred_element_type=jnp.float32) + bc_ref[...]


def _final(un_parts, den_parts, sc, sh, wc, bc):
    return pl.pallas_call(
        _final_body,
        grid=(N // BLK,),
        in_specs=[
            pl.BlockSpec((2, BLK, D), lambda i: (0, i, 0)),
            pl.BlockSpec((NW, BLK, 1), lambda i: (0, i, 0)),
            pl.BlockSpec((1, D), lambda i: (0, 0)),
            pl.BlockSpec((1, D), lambda i: (0, 0)),
            pl.BlockSpec((D, 2), lambda i: (0, 0)),
            pl.BlockSpec((1, 2), lambda i: (0, 0)),
        ],
        out_specs=pl.BlockSpec((BLK, 2), lambda i: (i, 0)),
        out_shape=jax.ShapeDtypeStruct((N, 2), jnp.float32),
    )(un_parts, den_parts, sc, sh, wc, bc)


CHK_A = 400


def _make_softmax(H):
    HN = H * N
    EPT = E // NW

    def body(as_hbm, ad_hbm, src_hbm, dst_hbm, ex_hbm, den_hbm,
             as_v, ad_v, den_v, src_v, dst_v, ex_v):
        core = lax.axis_index("c")
        sub = lax.axis_index("s")
        wid = core * NS + sub
        pltpu.sync_copy(as_hbm, as_v)
        pltpu.sync_copy(ad_hbm, ad_v)
        z16 = jnp.zeros((16,), jnp.float32)

        def zbody(i, _):
            den_v[pl.ds(i * 16, 16)] = z16
            return 0
        lax.fori_loop(0, HN // 16, zbody, 0)

        ebase = wid * EPT

        def chunk(kk, _):
            base = ebase + kk * CHK_A
            pltpu.sync_copy(src_hbm.at[pl.ds(base, CHK_A)], src_v)
            pltpu.sync_copy(dst_hbm.at[pl.ds(base, CHK_A)], dst_v)

            def grp(g, _):
                sv = src_v[pl.ds(g * 16, 16)] * H
                dv = dst_v[pl.ds(g * 16, 16)] * H
                for h in range(H):
                    a = plsc.load_gather(as_v, [sv + h])
                    b = plsc.load_gather(ad_v, [dv + h])
                    e = a + b
                    e = jnp.where(e >= 0, e, 0.2 * e)
                    ex = jnp.exp(e)
                    ex_v[pl.ds(h * CHK_A + g * 16, 16)] = ex
                    plsc.addupdate_scatter(den_v, [dv + h], ex)
                return 0
            lax.fori_loop(0, CHK_A // 16, grp, 0)
            for h in range(H):
                pltpu.sync_copy(ex_v.at[pl.ds(h * CHK_A, CHK_A)],
                                ex_hbm.at[pl.ds(h * E + base, CHK_A)])
            return 0
        lax.fori_loop(0, EPT // CHK_A, chunk, 0)
        pltpu.sync_copy(den_v, den_hbm.at[pl.ds(wid * HN, HN)])

    return pl.kernel(
        body,
        out_type=[
            jax.ShapeDtypeStruct((H * E,), jnp.float32),
            jax.ShapeDtypeStruct((NW * HN,), jnp.float32),
        ],
        mesh=_sc_mesh(),
        compiler_params=pltpu.CompilerParams(needs_layout_passes=False),
        scratch_types=[
            pltpu.VMEM((HN,), jnp.float32),
            pltpu.VMEM((HN,), jnp.float32),
            pltpu.VMEM((HN,), jnp.float32),
            pltpu.VMEM((CHK_A,), jnp.int32),
            pltpu.VMEM((CHK_A,), jnp.int32),
            pltpu.VMEM((H * CHK_A,), jnp.float32),
        ],
    )


K_B = 80
ZR = 128
NP = 10240
NRT = NP // NS


def _make_agg(H):
    HPC = 2 if H == 4 else 1
    EPT = E // NS if H == 4 else E // NW
    out_rows = H * NP if H == 4 else NC * NP

    NCH = EPT // K_B

    def body(hh_hbm, src_hbm, dst_hbm, ex_hbm, un_hbm, acc, zrow,
             src0, src1, dst0, dst1, sidx0, sidx1, ex0, ex1, rows0, rows1,
             semg0, semg1, sems0, sems1):
        core = lax.axis_index("c")
        sub = lax.axis_index("s")
        src_v = (src0, src1)
        dst_v = (dst0, dst1)
        sidx_v = (sidx0, sidx1) if H == 4 else (src0, src1)
        ex_v = (ex0, ex1)
        rows_v = (rows0, rows1)
        semg = (semg0, semg1)
        sems = (sems0, sems1)
        z16 = jnp.zeros((16,), jnp.float32)

        def zb(i, _):
            for cc in range(8):
                zrow[i, pl.ds(cc * 16, 16)] = z16
            return 0
        lax.fori_loop(0, ZR, zb, 0)

        r0 = sub * NRT
        for hh in range(HPC):
            if H == 4:
                h = core * HPC + hh
                ebase = sub * EPT
                orow = h * NP + r0
            else:
                h = 0
                ebase = (core * NS + sub) * EPT
                orow = core * NP + r0
            for q in range(NRT // ZR):
                pltpu.sync_copy(zrow, acc.at[pl.ds(r0 + q * ZR, ZR)])
            plsc.subcore_barrier()

            def fetch(kk, b):
                base = ebase + kk * K_B
                pltpu.sync_copy(src_hbm.at[pl.ds(base, K_B)], src_v[b])
                pltpu.sync_copy(dst_hbm.at[pl.ds(base, K_B)], dst_v[b])
                pltpu.sync_copy(ex_hbm.at[pl.ds(h * E + base, K_B)], ex_v[b])
                if H == 4:
                    def adj(g, _):
                        sl = pl.ds(g * 16, 16)
                        sidx_v[b][sl] = src_v[b][sl] + h * N
                        return 0
                    lax.fori_loop(0, K_B // 16, adj, 0)
                pltpu.async_copy(hh_hbm.at[sidx_v[b]], rows_v[b], semg[b])

            def wait_gather(b):
                pltpu.make_async_copy(hh_hbm.at[sidx_v[b]], rows_v[b],
                                      semg[b]).wait()

            def do_scale(b):
                def grp(g, _):
                    base16 = g * 16
                    for l in range(16):
                        j = base16 + l
                        eb = plsc.load_gather(
                            ex_v[b], [jnp.full((16,), j, jnp.int32)])
                        for cc in range(8):
                            sl = pl.ds(cc * 16, 16)
                            rows_v[b][j, sl] = rows_v[b][j, sl] * eb
                    return 0
                lax.fori_loop(0, K_B // 16, grp, 0)

            def issue_scatter(b):
                pltpu.async_copy(rows_v[b], acc.at[dst_v[b]], sems[b],
                                 add=True)

            def wait_scatter(b):
                pltpu.make_async_copy(rows_v[b], acc.at[dst_v[b]],
                                      sems[b]).wait()

            fetch(0, 0)

            def pair(i, _):
                k0 = i * 2
                for b in range(2):
                    kk = k0 + b
                    nb = 1 - b

                    @pl.when(kk + 1 < NCH)
                    def _():
                        @pl.when(kk >= 1)
                        def _():
                            wait_scatter(nb)
                        fetch(kk + 1, nb)
                    wait_gather(b)
                    do_scale(b)
                    issue_scatter(b)
                return 0
            lax.fori_loop(0, NCH // 2, pair, 0)
            if NCH % 2 == 1:
                wait_gather(0)
                do_scale(0)
                issue_scatter(0)
            wait_scatter((NCH - 2) % 2)
            wait_scatter((NCH - 1) % 2)
            plsc.subcore_barrier()
            pltpu.sync_copy(acc.at[pl.ds(r0, NRT)],
                            un_hbm.at[pl.ds(orow, NRT)])

    return pl.kernel(
        body,
        out_type=jax.ShapeDtypeStruct((out_rows, D), jnp.float32),
        mesh=_sc_mesh(),
        compiler_params=pltpu.CompilerParams(needs_layout_passes=False),
        scratch_types=[
            pltpu.VMEM_SHARED((NP, D), jnp.float32),
            pltpu.VMEM((ZR, D), jnp.float32),
            pltpu.VMEM((K_B,), jnp.int32),
            pltpu.VMEM((K_B,), jnp.int32),
            pltpu.VMEM((K_B,), jnp.int32),
            pltpu.VMEM((K_B,), jnp.int32),
            pltpu.VMEM((K_B,), jnp.int32),
            pltpu.VMEM((K_B,), jnp.int32),
            pltpu.VMEM((K_B,), jnp.float32),
            pltpu.VMEM((K_B,), jnp.float32),
            pltpu.VMEM((K_B, D), jnp.float32),
            pltpu.VMEM((K_B, D), jnp.float32),
            pltpu.SemaphoreType.DMA,
            pltpu.SemaphoreType.DMA,
            pltpu.SemaphoreType.DMA,
            pltpu.SemaphoreType.DMA,
        ],
    )


_softmax4 = _make_softmax(4)
_softmax1 = _make_softmax(1)
_agg4 = _make_agg(4)
_agg1 = _make_agg(1)


def _att_mat(att):
    H, C = att.shape
    mask = jnp.kron(jnp.eye(H, dtype=att.dtype), jnp.ones((C, 1), att.dtype))
    return mask * att.reshape(H * C, 1)


def _fold_bn(b, g, bb, rm, rv, eps=1e-5):
    s = g / jnp.sqrt(rv + eps)
    return s.reshape(1, -1), ((b - rm) * s + bb).reshape(1, -1)


def kernel(x, edge_index,
           W1, att_src1, att_dst1, b1, bn1_g, bn1_b, bn1_rm, bn1_rv,
           W2, att_src2, att_dst2, b2, bn2_g, bn2_b, bn2_rm, bn2_rv,
           W3, att_src3, att_dst3, b3, bn3_g, bn3_b, bn3_rm, bn3_rv,
           Wc, bc):
    src = edge_index[0]
    dst = edge_index[1]
    ms1, md1 = _att_mat(att_src1), _att_mat(att_dst1)
    ms2, md2 = _att_mat(att_src2), _att_mat(att_dst2)
    ms3, md3 = _att_mat(att_src3), _att_mat(att_dst3)
    sc1, sh1 = _fold_bn(b1, bn1_g, bn1_b, bn1_rm, bn1_rv)
    sc2, sh2 = _fold_bn(b2, bn2_g, bn2_b, bn2_rm, bn2_rv)
    sc3, sh3 = _fold_bn(b3, bn3_g, bn3_b, bn3_rm, bn3_rv)

    hh1, as1, ad1 = _dense_in(x, W1.T, ms1, md1)
    ex1, den1 = _softmax4(as1.reshape(-1), ad1.reshape(-1), src, dst)
    un1 = _agg4(hh1.reshape(4 * N, D), src, dst, ex1)
    hh2, as2, ad2 = _finish(un1.reshape(4, NP, D), den1.reshape(NW, N, 4),
                            sc1, sh1, W2.T, ms2, md2, 4)
    ex2, den2 = _softmax4(as2.reshape(-1), ad2.reshape(-1), src, dst)
    un2 = _agg4(hh2.reshape(4 * N, D), src, dst, ex2)
    hh3, as3, ad3 = _finish(un2.reshape(4, NP, D), den2.reshape(NW, N, 4),
                            sc2, sh2, W3.T, ms3, md3, 1)
    ex3, den3 = _softmax1(as3.reshape(-1), ad3.reshape(-1), src, dst)
    un3 = _agg1(hh3, src, dst, ex3)
    out = _final(un3.reshape(2, NP, D), den3.reshape(NW, N, 1),
                 sc3, sh3, Wc.T, bc.reshape(1, 2))
    return out


# speedup vs baseline: 25.0510x; 1.3572x over previous
import functools

import jax
import jax.numpy as jnp
from jax import lax
from jax.experimental import pallas as pl
from jax.experimental.pallas import tpu as pltpu
from jax.experimental.pallas import tpu_sc as plsc

N = 10000
E = 320000
D = 128
NC = 2
NS = 16
NW = NC * NS
BLK = 400


def _sc_mesh():
    return plsc.VectorSubcoreMesh(
        core_axis_name="c", subcore_axis_name="s", num_cores=NC, num_subcores=NS
    )


def _dense_in_body(x_ref, w_ref, ms_ref, md_ref, hh_ref, as_ref, ad_ref):
    hb = jnp.dot(x_ref[...], w_ref[...], preferred_element_type=jnp.float32)
    for h in range(4):
        hh_ref[h] = hb[:, h * D:(h + 1) * D]
    as_ref[...] = jnp.dot(hb, ms_ref[...], preferred_element_type=jnp.float32)
    ad_ref[...] = jnp.dot(hb, md_ref[...], preferred_element_type=jnp.float32)


def _dense_in(x, w, ms, md):
    return pl.pallas_call(
        _dense_in_body,
        grid=(N // BLK,),
        in_specs=[
            pl.BlockSpec((BLK, D), lambda i: (i, 0)),
            pl.BlockSpec((D, 4 * D), lambda i: (0, 0)),
            pl.BlockSpec((4 * D, 4), lambda i: (0, 0)),
            pl.BlockSpec((4 * D, 4), lambda i: (0, 0)),
        ],
        out_specs=[
            pl.BlockSpec((4, BLK, D), lambda i: (0, i, 0)),
            pl.BlockSpec((BLK, 4), lambda i: (i, 0)),
            pl.BlockSpec((BLK, 4), lambda i: (i, 0)),
        ],
        out_shape=[
            jax.ShapeDtypeStruct((4, N, D), jnp.float32),
            jax.ShapeDtypeStruct((N, 4), jnp.float32),
            jax.ShapeDtypeStruct((N, 4), jnp.float32),
        ],
    )(x, w, ms, md)


def _make_finish_body(nh_next):
    def body(un_ref, den_ref, sc_ref, sh_ref, w_ref, ms_ref, md_ref,
             hh_ref, as_ref, ad_ref):
        den = jnp.sum(den_ref[...], axis=0)
        parts = [un_ref[h] / (den[:, h:h + 1] + 1e-16) for h in range(4)]
        xb = jnp.concatenate(parts, axis=1)
        xb = xb * sc_ref[...] + sh_ref[...]
        xb = jnp.where(xb > 0, xb, jnp.exp(xb) - 1.0)
        hb = jnp.dot(xb, w_ref[...], preferred_element_type=jnp.float32)
        if nh_next == 4:
            for h in range(4):
                hh_ref[h] = hb[:, h * D:(h + 1) * D]
        else:
            hh_ref[...] = hb
        as_ref[...] = jnp.dot(hb, ms_ref[...], preferred_element_type=jnp.float32)
        ad_ref[...] = jnp.dot(hb, md_ref[...], preferred_element_type=jnp.float32)
    return body


def _finish(un, den_parts, sc, sh, w, ms, md, nh_next):
    wcols = w.shape[1]
    if nh_next == 4:
        hh_spec = pl.BlockSpec((4, BLK, D), lambda i: (0, i, 0))
        hh_shape = jax.ShapeDtypeStruct((4, N, D), jnp.float32)
    else:
        hh_spec = pl.BlockSpec((BLK, D), lambda i: (i, 0))
        hh_shape = jax.ShapeDtypeStruct((N, D), jnp.float32)
    return pl.pallas_call(
        _make_finish_body(nh_next),
        grid=(N // BLK,),
        in_specs=[
            pl.BlockSpec((4, BLK, D), lambda i: (0, i, 0)),
            pl.BlockSpec((NW, BLK, 4), lambda i: (0, i, 0)),
            pl.BlockSpec((1, 4 * D), lambda i: (0, 0)),
            pl.BlockSpec((1, 4 * D), lambda i: (0, 0)),
            pl.BlockSpec((4 * D, wcols), lambda i: (0, 0)),
            pl.BlockSpec((wcols, nh_next), lambda i: (0, 0)),
            pl.BlockSpec((wcols, nh_next), lambda i: (0, 0)),
        ],
        out_specs=[
            hh_spec,
            pl.BlockSpec((BLK, nh_next), lambda i: (i, 0)),
            pl.BlockSpec((BLK, nh_next), lambda i: (i, 0)),
        ],
        out_shape=[
            hh_shape,
            jax.ShapeDtypeStruct((N, nh_next), jnp.float32),
            jax.ShapeDtypeStruct((N, nh_next), jnp.float32),
        ],
    )(un, den_parts, sc, sh, w, ms, md)


def _final_body(un_ref, den_ref, sc_ref, sh_ref, wc_ref, bc_ref, out_ref):
    u = un_ref[0] + un_ref[1]
    den = jnp.sum(den_ref[...], axis=0)
    x = u / (den + 1e-16)
    x = x * sc_ref[...] + sh_ref[...]
    x = jnp.where(x > 0, x, jnp.exp(x) - 1.0)
    out_ref[...] = jnp.dot(x, wc_ref[...], preferred_element_type=jnp.float32) + bc_ref[...]


def _final(un_parts, den_parts, sc, sh, wc, bc):
    return pl.pallas_call(
        _final_body,
        grid=(N // BLK,),
        in_specs=[
            pl.BlockSpec((2, BLK, D), lambda i: (0, i, 0)),
            pl.BlockSpec((NW, BLK, 1), lambda i: (0, i, 0)),
            pl.BlockSpec((1, D), lambda i: (0, 0)),
            pl.BlockSpec((1, D), lambda i: (0, 0)),
            pl.BlockSpec((D, 2), lambda i: (0, 0)),
            pl.BlockSpec((1, 2), lambda i: (0, 0)),
        ],
        out_specs=pl.BlockSpec((BLK, 2), lambda i: (i, 0)),
        out_shape=jax.ShapeDtypeStruct((N, 2), jnp.float32),
    )(un_parts, den_parts, sc, sh, wc, bc)


CHK_A = 400


def _make_softmax(H):
    HN = H * N
    EPT = E // NW

    def body(as_hbm, ad_hbm, src_hbm, dst_hbm, ex_hbm, den_hbm,
             as_v, ad_v, den_v, src_v, dst_v, ex_v):
        core = lax.axis_index("c")
        sub = lax.axis_index("s")
        wid = core * NS + sub
        pltpu.sync_copy(as_hbm, as_v)
        pltpu.sync_copy(ad_hbm, ad_v)
        z16 = jnp.zeros((16,), jnp.float32)

        def zbody(i, _):
            den_v[pl.ds(i * 16, 16)] = z16
            return 0
        lax.fori_loop(0, HN // 16, zbody, 0)

        ebase = wid * EPT

        def chunk(kk, _):
            base = ebase + kk * CHK_A
            pltpu.sync_copy(src_hbm.at[pl.ds(base, CHK_A)], src_v)
            pltpu.sync_copy(dst_hbm.at[pl.ds(base, CHK_A)], dst_v)

            def grp(g, _):
                sv = src_v[pl.ds(g * 16, 16)] * H
                dv = dst_v[pl.ds(g * 16, 16)] * H
                for h in range(H):
                    a = plsc.load_gather(as_v, [sv + h])
                    b = plsc.load_gather(ad_v, [dv + h])
                    e = a + b
                    e = jnp.where(e >= 0, e, 0.2 * e)
                    ex = jnp.exp(e)
                    ex_v[pl.ds(h * CHK_A + g * 16, 16)] = ex
                    plsc.addupdate_scatter(den_v, [dv + h], ex)
                return 0
            lax.fori_loop(0, CHK_A // 16, grp, 0)
            for h in range(H):
                pltpu.sync_copy(ex_v.at[pl.ds(h * CHK_A, CHK_A)],
                                ex_hbm.at[pl.ds(h * E + base, CHK_A)])
            return 0
        lax.fori_loop(0, EPT // CHK_A, chunk, 0)
        pltpu.sync_copy(den_v, den_hbm.at[pl.ds(wid * HN, HN)])

    return pl.kernel(
        body,
        out_type=[
            jax.ShapeDtypeStruct((H * E,), jnp.float32),
            jax.ShapeDtypeStruct((NW * HN,), jnp.float32),
        ],
        mesh=_sc_mesh(),
        compiler_params=pltpu.CompilerParams(needs_layout_passes=False),
        scratch_types=[
            pltpu.VMEM((HN,), jnp.float32),
            pltpu.VMEM((HN,), jnp.float32),
            pltpu.VMEM((HN,), jnp.float32),
            pltpu.VMEM((CHK_A,), jnp.int32),
            pltpu.VMEM((CHK_A,), jnp.int32),
            pltpu.VMEM((H * CHK_A,), jnp.float32),
        ],
    )


K_B = 80
ZR = 128
NP = 10240
NRT = NP // NS


def _make_agg(H):
    HPC = 2 if H == 4 else 1
    EPT = E // NS if H == 4 else E // NW
    out_rows = H * NP if H == 4 else NC * NP

    NCH = EPT // K_B

    def body(hh_hbm, src_hbm, dst_hbm, ex_hbm, un_hbm, acc, zrow,
             src0, src1, dst0, dst1, sidx0, sidx1, ex0, ex1, rows0, rows1,
             semg0, semg1, sems0, sems1, semsm):
        core = lax.axis_index("c")
        sub = lax.axis_index("s")
        src_v = (src0, src1)
        dst_v = (dst0, dst1)
        sidx_v = (sidx0, sidx1) if H == 4 else (src0, src1)
        ex_v = (ex0, ex1)
        rows_v = (rows0, rows1)
        semg = (semg0, semg1)
        sems = (sems0, sems1)
        z16 = jnp.zeros((16,), jnp.float32)

        def zb(i, _):
            for cc in range(8):
                zrow[i, pl.ds(cc * 16, 16)] = z16
            return 0
        lax.fori_loop(0, ZR, zb, 0)

        r0 = sub * NRT
        for hh in range(HPC):
            if H == 4:
                h = core * HPC + hh
                ebase = sub * EPT
                orow = h * NP + r0
            else:
                h = 0
                ebase = (core * NS + sub) * EPT
                orow = core * NP + r0
            for q in range(NRT // ZR):
                pltpu.sync_copy(zrow, acc.at[pl.ds(r0 + q * ZR, ZR)])
            plsc.subcore_barrier()

            def fetch(kk, b):
                base = ebase + kk * K_B
                a1 = pltpu.async_copy(src_hbm.at[pl.ds(base, K_B)],
                                      src_v[b], semsm)
                a2 = pltpu.async_copy(dst_hbm.at[pl.ds(base, K_B)],
                                      dst_v[b], semsm)
                a3 = pltpu.async_copy(ex_hbm.at[pl.ds(h * E + base, K_B)],
                                      ex_v[b], semsm)
                a1.wait()
                a2.wait()
                a3.wait()
                if H == 4:
                    def adj(g, _):
                        sl = pl.ds(g * 16, 16)
                        sidx_v[b][sl] = src_v[b][sl] + h * N
                        return 0
                    lax.fori_loop(0, K_B // 16, adj, 0)
                pltpu.async_copy(hh_hbm.at[sidx_v[b]], rows_v[b], semg[b])

            def wait_gather(b):
                pltpu.make_async_copy(hh_hbm.at[sidx_v[b]], rows_v[b],
                                      semg[b]).wait()

            def do_scale(b):
                def grp(g, _):
                    base16 = g * 16
                    for l in range(16):
                        j = base16 + l
                        eb = plsc.load_gather(
                            ex_v[b], [jnp.full((16,), j, jnp.int32)])
                        for cc in range(8):
                            sl = pl.ds(cc * 16, 16)
                            rows_v[b][j, sl] = rows_v[b][j, sl] * eb
                    return 0
                lax.fori_loop(0, K_B // 16, grp, 0)

            def issue_scatter(b):
                pltpu.async_copy(rows_v[b], acc.at[dst_v[b]], sems[b],
                                 add=True)

            def wait_scatter(b):
                pltpu.make_async_copy(rows_v[b], acc.at[dst_v[b]],
                                      sems[b]).wait()

            fetch(0, 0)

            def pair(i, _):
                k0 = i * 2
                for b in range(2):
                    kk = k0 + b
                    nb = 1 - b

                    @pl.when(kk + 1 < NCH)
                    def _():
                        @pl.when(kk >= 1)
                        def _():
                            wait_scatter(nb)
                        fetch(kk + 1, nb)
                    wait_gather(b)
                    do_scale(b)
                    issue_scatter(b)
                return 0
            lax.fori_loop(0, NCH // 2, pair, 0)
            if NCH % 2 == 1:
                wait_gather(0)
                do_scale(0)
                issue_scatter(0)
            wait_scatter((NCH - 2) % 2)
            wait_scatter((NCH - 1) % 2)
            plsc.subcore_barrier()
            pltpu.sync_copy(acc.at[pl.ds(r0, NRT)],
                            un_hbm.at[pl.ds(orow, NRT)])

    return pl.kernel(
        body,
        out_type=jax.ShapeDtypeStruct((out_rows, D), jnp.float32),
        mesh=_sc_mesh(),
        compiler_params=pltpu.CompilerParams(needs_layout_passes=False),
        scratch_types=[
            pltpu.VMEM_SHARED((NP, D), jnp.float32),
            pltpu.VMEM((ZR, D), jnp.float32),
            pltpu.VMEM((K_B,), jnp.int32),
            pltpu.VMEM((K_B,), jnp.int32),
            pltpu.VMEM((K_B,), jnp.int32),
            pltpu.VMEM((K_B,), jnp.int32),
            pltpu.VMEM((K_B,), jnp.int32),
            pltpu.VMEM((K_B,), jnp.int32),
            pltpu.VMEM((K_B,), jnp.float32),
            pltpu.VMEM((K_B,), jnp.float32),
            pltpu.VMEM((K_B, D), jnp.float32),
            pltpu.VMEM((K_B, D), jnp.float32),
            pltpu.SemaphoreType.DMA,
            pltpu.SemaphoreType.DMA,
            pltpu.SemaphoreType.DMA,
            pltpu.SemaphoreType.DMA,
            pltpu.SemaphoreType.DMA,
        ],
    )


_softmax4 = _make_softmax(4)
_softmax1 = _make_softmax(1)
_agg4 = _make_agg(4)
_agg1 = _make_agg(1)


def _att_mat(att):
    H, C = att.shape
    mask = jnp.kron(jnp.eye(H, dtype=att.dtype), jnp.ones((C, 1), att.dtype))
    return mask * att.reshape(H * C, 1)


def _fold_bn(b, g, bb, rm, rv, eps=1e-5):
    s = g / jnp.sqrt(rv + eps)
    return s.reshape(1, -1), ((b - rm) * s + bb).reshape(1, -1)


def kernel(x, edge_index,
           W1, att_src1, att_dst1, b1, bn1_g, bn1_b, bn1_rm, bn1_rv,
           W2, att_src2, att_dst2, b2, bn2_g, bn2_b, bn2_rm, bn2_rv,
           W3, att_src3, att_dst3, b3, bn3_g, bn3_b, bn3_rm, bn3_rv,
           Wc, bc):
    src = edge_index[0]
    dst = edge_index[1]
    ms1, md1 = _att_mat(att_src1), _att_mat(att_dst1)
    ms2, md2 = _att_mat(att_src2), _att_mat(att_dst2)
    ms3, md3 = _att_mat(att_src3), _att_mat(att_dst3)
    sc1, sh1 = _fold_bn(b1, bn1_g, bn1_b, bn1_rm, bn1_rv)
    sc2, sh2 = _fold_bn(b2, bn2_g, bn2_b, bn2_rm, bn2_rv)
    sc3, sh3 = _fold_bn(b3, bn3_g, bn3_b, bn3_rm, bn3_rv)

    hh1, as1, ad1 = _dense_in(x, W1.T, ms1, md1)
    ex1, den1 = _softmax4(as1.reshape(-1), ad1.reshape(-1), src, dst)
    un1 = _agg4(hh1.reshape(4 * N, D), src, dst, ex1)
    hh2, as2, ad2 = _finish(un1.reshape(4, NP, D), den1.reshape(NW, N, 4),
                            sc1, sh1, W2.T, ms2, md2, 4)
    ex2, den2 = _softmax4(as2.reshape(-1), ad2.reshape(-1), src, dst)
    un2 = _agg4(hh2.reshape(4 * N, D), src, dst, ex2)
    hh3, as3, ad3 = _finish(un2.reshape(4, NP, D), den2.reshape(NW, N, 4),
                            sc2, sh2, W3.T, ms3, md3, 1)
    ex3, den3 = _softmax1(as3.reshape(-1), ad3.reshape(-1), src, dst)
    un3 = _agg1(hh3, src, dst, ex3)
    out = _final(un3.reshape(2, NP, D), den3.reshape(NW, N, 1),
                 sc3, sh3, Wc.T, bc.reshape(1, 2))
    return out
